# baseline (speedup 1.0000x reference)
import functools

import numpy as np

import jax
import jax.numpy as jnp
from jax import lax
from jax.experimental import pallas as pl
from jax.experimental.pallas import tpu as pltpu
from jax.experimental.pallas import tpu_sc as plsc

N = 10000
E = 320000
D = 128

NC = 2
NS = 16
NW = NC * NS
EPW = E // NW
C = 80
NCHUNK = EPW // C
RPT = N // NS


ACCW = 144


def _proj_body(x_ref, wq, bq, wk, bk, wv, bv, ws, bs, q_out, k_out, v_out,
               s_out):
    xb = x_ref[...]
    scale = jnp.float32(D ** (-0.5))
    q_out[...] = ((xb @ wq[...] + bq[...]) * scale).astype(jnp.bfloat16)
    k_out[...] = (xb @ wk[...] + bk[...]).astype(jnp.bfloat16)
    v_out[...] = xb @ wv[...] + bv[...]
    s_out[...] = xb @ ws[...] + bs[...]


def _projections(x, Wq, bq, Wk, bk, Wvp, bvp, Wself, bself):
    RB = 1000
    grid = (N // RB,)
    row_spec = pl.BlockSpec((RB, D), lambda i: (i, 0))
    vrow_spec = pl.BlockSpec((RB, ACCW), lambda i: (i, 0))
    w_spec = pl.BlockSpec((D, D), lambda i: (0, 0))
    wv_spec = pl.BlockSpec((D, ACCW), lambda i: (0, 0))
    b_spec = pl.BlockSpec((1, D), lambda i: (0, 0))
    bv_spec = pl.BlockSpec((1, ACCW), lambda i: (0, 0))
    return pl.pallas_call(
        _proj_body,
        grid=grid,
        in_specs=[row_spec, w_spec, b_spec, w_spec, b_spec, wv_spec, bv_spec,
                  w_spec, b_spec],
        out_specs=[row_spec, row_spec, vrow_spec, row_spec],
        out_shape=[
            jax.ShapeDtypeStruct((N, D), jnp.bfloat16),
            jax.ShapeDtypeStruct((N, D), jnp.bfloat16),
            jax.ShapeDtypeStruct((N, ACCW), jnp.float32),
            jax.ShapeDtypeStruct((N, D), jnp.float32),
        ],
    )(x, Wq, bq.reshape(1, D), Wk, bk.reshape(1, D), Wvp,
      bvp.reshape(1, ACCW), Wself, bself.reshape(1, D))


def _edge_body(q_hbm, k_hbm, v_hbm, idx_hbm, webe_hbm,
               out_hbm,
               acc, idx3, qb, kb, vb, webev,
               gs0, gs1, ks, isem):
    cid = lax.axis_index("c")
    sid = lax.axis_index("s")
    wid = sid * NC + cid
    gsems = (gs0, gs1)

    def zrow(i, carry):
        for j in range(ACCW // 16):
            vb[0, i, pl.ds(j * 16, 16)] = jnp.zeros((16,), jnp.float32)
        return carry
    lax.fori_loop(0, C, zrow, 0)

    rbase = sid * RPT
    zcps = []
    for t in range(7):
        zcps.append(pltpu.async_copy(
            vb.at[0], acc.at[pl.ds(rbase + t * C, C)], gs0))
    zcps.append(pltpu.async_copy(
        vb.at[0, pl.ds(0, 65)], acc.at[pl.ds(rbase + 7 * C, 65)], gs0))
    for cp in zcps:
        cp.wait()
    plsc.subcore_barrier()

    pltpu.sync_copy(webe_hbm, webev)
    we_v = webev[0, :]
    be_v = webev[1, :]

    cbase = wid * NCHUNK

    def issue_idx(t, pair_no):
        pltpu.async_copy(idx_hbm.at[pl.ds(cbase + 2 * pair_no, 2)],
                         idx3.at[t], isem)

    def wait_idx(t, pair_no):
        pltpu.make_async_copy(idx_hbm.at[pl.ds(cbase + 2 * pair_no, 2)],
                              idx3.at[t], isem).wait()

    def issue_qv(s, t, r):
        pltpu.async_copy(q_hbm.at[idx3.at[t, r, 1]], qb.at[s], gsems[s])
        pltpu.async_copy(v_hbm.at[idx3.at[t, r, 0]], vb.at[s], gsems[s])

    def wait_qv(s, t, r):
        pltpu.make_async_copy(q_hbm.at[idx3.at[t, r, 1]], qb.at[s],
                              gsems[s]).wait()
        pltpu.make_async_copy(v_hbm.at[idx3.at[t, r, 0]], vb.at[s],
                              gsems[s]).wait()

    def issue_k(t, r):
        pltpu.async_copy(k_hbm.at[idx3.at[t, r, 0]], kb, ks)

    def wait_k(t, r):
        pltpu.make_async_copy(k_hbm.at[idx3.at[t, r, 0]], kb, ks).wait()

    def scores_and_scale(s, t, r):
        def grp(g, carry):
            ewi = idx3[t, r, 2, pl.ds(g * 16, 16)]
            bias16 = plsc.bitcast(ewi, jnp.float32) * we_v + be_v
            for u in range(16):
                e = g * 16 + u
                a = None
                for h in range(4):
                    q32 = qb[s, e, pl.ds(h * 32, 32)]
                    k32 = kb[e, pl.ds(h * 32, 32)]
                    term = q32 * k32
                    a = term if a is None else a + term
                aa, az = plsc.unpack(a, format=plsc.PackFormat.INTERLEAVED)
                sc = jnp.sum(aa + az) + bias16[u]
                exv = jnp.exp(jnp.broadcast_to(sc, (16,)))
                for w in range(ACCW // 16):
                    vb[s, e, pl.ds(w * 16, 16)] = (
                        vb[s, e, pl.ds(w * 16, 16)] * exv)
            return carry
        lax.fori_loop(0, C // 16, grp, 0)

    def scatter(s, t, r):
        pltpu.sync_copy(vb.at[s], acc.at[idx3.at[t, r, 1]], add=True)

    pltpu.sync_copy(idx_hbm.at[pl.ds(cbase, 2)], idx3.at[0])
    issue_qv(0, 0, 0)
    issue_k(0, 0)

    def quad(i, carry):
        issue_idx(1, 2 * i + 1)
        issue_qv(1, 0, 1)
        wait_qv(0, 0, 0)
        wait_k(0, 0)
        scores_and_scale(0, 0, 0)
        issue_k(0, 1)
        scatter(0, 0, 0)
        wait_idx(1, 2 * i + 1)
        issue_qv(0, 1, 0)
        wait_qv(1, 0, 1)
        wait_k(0, 1)
        scores_and_scale(1, 0, 1)
        issue_k(1, 0)
        scatter(1, 0, 1)
        issue_idx(0, 2 * i + 2)
        issue_qv(1, 1, 1)
        wait_qv(0, 1, 0)
        wait_k(1, 0)
        scores_and_scale(0, 1, 0)
        issue_k(1, 1)
        scatter(0, 1, 0)
        wait_idx(0, 2 * i + 2)
        issue_qv(0, 0, 0)
        wait_qv(1, 1, 1)
        wait_k(1, 1)
        scores_and_scale(1, 1, 1)
        issue_k(0, 0)
        scatter(1, 1, 1)
        return carry
    lax.fori_loop(0, (NCHUNK - 1) // 4, quad, 0)

    wait_qv(0, 0, 0)
    wait_k(0, 0)
    scores_and_scale(0, 0, 0)
    scatter(0, 0, 0)

    plsc.subcore_barrier()
    ocps = []
    for t in range(7):
        sl = pl.ds(rbase + t * C, C)
        ocps.append(pltpu.async_copy(acc.at[sl], out_hbm.at[cid, sl], gs0))
    sl = pl.ds(rbase + 7 * C, 65)
    ocps.append(pltpu.async_copy(acc.at[sl], out_hbm.at[cid, sl], gs0))
    for cp in ocps:
        cp.wait()


def _edge_pass(q, k, v, idx_packed, webe):
    mesh = plsc.VectorSubcoreMesh(core_axis_name="c", subcore_axis_name="s")
    f = pl.kernel(
        _edge_body,
        out_type=jax.ShapeDtypeStruct((NC, N, ACCW), jnp.float32),
        mesh=mesh,
        compiler_params=pltpu.CompilerParams(
            needs_layout_passes=False, use_tc_tiling_on_sc=False),
        scratch_types=[
            pltpu.VMEM_SHARED((N, ACCW), jnp.float32),
            pltpu.VMEM((2, 2, 3, C), jnp.int32),
            pltpu.VMEM((2, C, D), jnp.bfloat16),
            pltpu.VMEM((C, D), jnp.bfloat16),
            pltpu.VMEM((2, C, ACCW), jnp.float32),
            pltpu.VMEM((2, 16), jnp.float32),
            pltpu.SemaphoreType.DMA,
            pltpu.SemaphoreType.DMA,
            pltpu.SemaphoreType.DMA,
            pltpu.SemaphoreType.DMA,
        ],
    )
    return f(q, k, v, idx_packed, webe)


def _ln(x, g, b, eps=1e-5):
    mu = jnp.mean(x, axis=-1, keepdims=True)
    var = jnp.mean((x - mu) * (x - mu), axis=-1, keepdims=True)
    return (x - mu) / jnp.sqrt(var + eps) * g + b


def _final_body(x_ref, so_ref, p_ref, g1, b1, g2, b2,
                wf1, bf1, wf2, bf2, out_ref):
    p = p_ref[0] + p_ref[1]
    agg = p[:, :D] / jnp.clip(p[:, D:D + 1], 1e-12, None)
    h = _ln(x_ref[...] + agg + so_ref[...], g1[...], b1[...])
    f = jax.nn.gelu(h @ wf1[...] + bf1[...]) @ wf2[...] + bf2[...]
    out_ref[...] = _ln(h + f, g2[...], b2[...])


def _final(x, selfo, parts, g1, b1, g2, b2, Wf1, bf1, Wf2, bf2):
    RB = 1000
    grid = (N // RB,)
    row_spec = pl.BlockSpec((RB, D), lambda i: (i, 0))
    p_spec = pl.BlockSpec((NC, RB, ACCW), lambda i: (0, i, 0))
    vec_spec = pl.BlockSpec((1, D), lambda i: (0, 0))
    vec2_spec = pl.BlockSpec((1, 2 * D), lambda i: (0, 0))
    w1_spec = pl.BlockSpec((D, 2 * D), lambda i: (0, 0))
    w2_spec = pl.BlockSpec((2 * D, D), lambda i: (0, 0))
    return pl.pallas_call(
        _final_body,
        grid=grid,
        in_specs=[row_spec, row_spec, p_spec,
                  vec_spec, vec_spec, vec_spec, vec_spec,
                  w1_spec, vec2_spec, w2_spec, vec_spec],
        out_specs=row_spec,
        out_shape=jax.ShapeDtypeStruct((N, D), jnp.float32),
    )(x, selfo, parts,
      g1.reshape(1, D), b1.reshape(1, D), g2.reshape(1, D), b2.reshape(1, D),
      Wf1, bf1.reshape(1, 2 * D), Wf2, bf2.reshape(1, D))


def kernel(x, edge_index, edge_weight, Wq, bq, Wk, bk, Wv, bv, Wself, bself,
           We, be, g1, b1, g2, b2, Wf1, bf1, Wf2, bf2):
    src = edge_index[0].astype(jnp.int32)
    dst = edge_index[1].astype(jnp.int32)
    ew = edge_weight.astype(jnp.float32)
    ew_bits = lax.bitcast_convert_type(ew, jnp.int32)

    idx_packed = jnp.stack(
        [src.reshape(NW * NCHUNK, C),
         dst.reshape(NW * NCHUNK, C),
         ew_bits.reshape(NW * NCHUNK, C)], axis=1)
    idx_packed = jnp.concatenate(
        [idx_packed, jnp.zeros((1, 3, C), jnp.int32)], axis=0)

    Wvp = jnp.concatenate([Wv, jnp.zeros((D, ACCW - D), jnp.float32)], axis=1)
    bvp = jnp.concatenate(
        [bv, jnp.ones((1,), jnp.float32), jnp.zeros((ACCW - D - 1,),
                                                    jnp.float32)])

    q, k, v, selfo = _projections(x, Wq, bq, Wk, bk, Wvp, bvp, Wself, bself)

    webe = jnp.stack([
        jnp.broadcast_to(We.reshape(()), (16,)),
        jnp.broadcast_to(be.reshape(()), (16,)),
    ]).astype(jnp.float32)

    parts = _edge_pass(q, k, v, idx_packed, webe)

    return _final(x, selfo, parts, g1, b1, g2, b2, Wf1, bf1, Wf2, bf2)


# speedup vs baseline: 1.0400x; 1.0400x over previous
import functools

import numpy as np

import jax
import jax.numpy as jnp
from jax import lax
from jax.experimental import pallas as pl
from jax.experimental.pallas import tpu as pltpu
from jax.experimental.pallas import tpu_sc as plsc

N = 10000
E = 320000
D = 128

NC = 2
NS = 16
NW = NC * NS
EPW = E // NW
C = 80
NCHUNK = EPW // C
RPT = N // NS


def _proj_body(x_ref, wq, bq, wk, bk, wv, bv, ws, bs, q_out, k_out, v_out,
               s_out):
    xb = x_ref[...]
    scale = jnp.float32(D ** (-0.5))
    q_out[...] = ((xb @ wq[...] + bq[...]) * scale).astype(jnp.bfloat16)
    k_out[...] = (xb @ wk[...] + bk[...]).astype(jnp.bfloat16)
    v_out[...] = xb @ wv[...] + bv[...]
    s_out[...] = xb @ ws[...] + bs[...]


def _projections(x, Wq, bq, Wk, bk, Wv, bv, Wself, bself):
    RB = 1000
    grid = (N // RB,)
    row_spec = pl.BlockSpec((RB, D), lambda i: (i, 0))
    w_spec = pl.BlockSpec((D, D), lambda i: (0, 0))
    b_spec = pl.BlockSpec((1, D), lambda i: (0, 0))
    return pl.pallas_call(
        _proj_body,
        grid=grid,
        in_specs=[row_spec, w_spec, b_spec, w_spec, b_spec, w_spec, b_spec,
                  w_spec, b_spec],
        out_specs=[row_spec, row_spec, row_spec, row_spec],
        out_shape=[
            jax.ShapeDtypeStruct((N, D), jnp.bfloat16),
            jax.ShapeDtypeStruct((N, D), jnp.bfloat16),
            jax.ShapeDtypeStruct((N, D), jnp.float32),
            jax.ShapeDtypeStruct((N, D), jnp.float32),
        ],
    )(x, Wq, bq.reshape(1, D), Wk, bk.reshape(1, D), Wv, bv.reshape(1, D),
      Wself, bself.reshape(1, D))


def _edge_body(q_hbm, k_hbm, v_hbm, idx_hbm, webe_hbm,
               out_hbm, outd_hbm,
               acc, accd, idx3, qb, kb, vb, obd, webev,
               gs0, gs1, ks, isem):
    cid = lax.axis_index("c")
    sid = lax.axis_index("s")
    wid = sid * NC + cid
    gsems = (gs0, gs1)

    def zrow(i, carry):
        for j in range(D // 16):
            vb[0, i, pl.ds(j * 16, 16)] = jnp.zeros((16,), jnp.float32)
        obd[0, i, :] = jnp.zeros((16,), jnp.float32)
        return carry
    lax.fori_loop(0, C, zrow, 0)

    rbase = sid * RPT
    zcps = []
    for t in range(7):
        zcps.append(pltpu.async_copy(
            vb.at[0], acc.at[pl.ds(rbase + t * C, C)], gs0))
        zcps.append(pltpu.async_copy(
            obd.at[0], accd.at[pl.ds(rbase + t * C, C)], gs0))
    zcps.append(pltpu.async_copy(
        vb.at[0, pl.ds(0, 65)], acc.at[pl.ds(rbase + 7 * C, 65)], gs0))
    zcps.append(pltpu.async_copy(
        obd.at[0, pl.ds(0, 65)], accd.at[pl.ds(rbase + 7 * C, 65)], gs0))
    for cp in zcps:
        cp.wait()
    plsc.subcore_barrier()

    pltpu.sync_copy(webe_hbm, webev)
    we_v = webev[0, :]
    be_v = webev[1, :]
    lane0 = lax.iota(jnp.int32, 16) == 0

    cbase = wid * NCHUNK

    def issue_idx(t, pair_no):
        pltpu.async_copy(idx_hbm.at[pl.ds(cbase + 2 * pair_no, 2)],
                         idx3.at[t], isem)

    def wait_idx(t, pair_no):
        pltpu.make_async_copy(idx_hbm.at[pl.ds(cbase + 2 * pair_no, 2)],
                              idx3.at[t], isem).wait()

    def issue_qv(s, t, r):
        pltpu.async_copy(q_hbm.at[idx3.at[t, r, 1]], qb.at[s], gsems[s])
        pltpu.async_copy(v_hbm.at[idx3.at[t, r, 0]], vb.at[s], gsems[s])

    def wait_qv(s, t, r):
        pltpu.make_async_copy(q_hbm.at[idx3.at[t, r, 1]], qb.at[s],
                              gsems[s]).wait()
        pltpu.make_async_copy(v_hbm.at[idx3.at[t, r, 0]], vb.at[s],
                              gsems[s]).wait()

    def issue_k(t, r):
        pltpu.async_copy(k_hbm.at[idx3.at[t, r, 0]], kb, ks)

    def wait_k(t, r):
        pltpu.make_async_copy(k_hbm.at[idx3.at[t, r, 0]], kb, ks).wait()

    def scores_and_scale(s, t, r):
        def grp(g, carry):
            ewi = idx3[t, r, 2, pl.ds(g * 16, 16)]
            bias16 = plsc.bitcast(ewi, jnp.float32) * we_v + be_v
            for u in range(16):
                e = g * 16 + u
                a = None
                for h in range(4):
                    q32 = qb[s, e, pl.ds(h * 32, 32)]
                    k32 = kb[e, pl.ds(h * 32, 32)]
                    term = q32 * k32
                    a = term if a is None else a + term
                aa, az = plsc.unpack(a, format=plsc.PackFormat.INTERLEAVED)
                sc = jnp.sum(aa + az) + bias16[u]
                exv = jnp.exp(jnp.broadcast_to(sc, (16,)))
                for w in range(D // 16):
                    vb[s, e, pl.ds(w * 16, 16)] = (
                        vb[s, e, pl.ds(w * 16, 16)] * exv)
                obd[s, e, :] = jnp.where(lane0, exv, jnp.float32(0.0))
            return carry
        lax.fori_loop(0, C // 16, grp, 0)

    def scatter(s, t, r):
        pltpu.sync_copy(vb.at[s], acc.at[idx3.at[t, r, 1]], add=True)
        pltpu.sync_copy(obd.at[s], accd.at[idx3.at[t, r, 1]], add=True)

    pltpu.sync_copy(idx_hbm.at[pl.ds(cbase, 2)], idx3.at[0])
    issue_qv(0, 0, 0)
    issue_k(0, 0)

    def quad(i, carry):
        issue_idx(1, 2 * i + 1)
        issue_qv(1, 0, 1)
        wait_qv(0, 0, 0)
        wait_k(0, 0)
        scores_and_scale(0, 0, 0)
        issue_k(0, 1)
        scatter(0, 0, 0)
        wait_idx(1, 2 * i + 1)
        issue_qv(0, 1, 0)
        wait_qv(1, 0, 1)
        wait_k(0, 1)
        scores_and_scale(1, 0, 1)
        issue_k(1, 0)
        scatter(1, 0, 1)
        issue_idx(0, 2 * i + 2)
        issue_qv(1, 1, 1)
        wait_qv(0, 1, 0)
        wait_k(1, 0)
        scores_and_scale(0, 1, 0)
        issue_k(1, 1)
        scatter(0, 1, 0)
        wait_idx(0, 2 * i + 2)
        issue_qv(0, 0, 0)
        wait_qv(1, 1, 1)
        wait_k(1, 1)
        scores_and_scale(1, 1, 1)
        issue_k(0, 0)
        scatter(1, 1, 1)
        return carry
    lax.fori_loop(0, (NCHUNK - 1) // 4, quad, 0)

    wait_qv(0, 0, 0)
    wait_k(0, 0)
    scores_and_scale(0, 0, 0)
    scatter(0, 0, 0)

    plsc.subcore_barrier()
    ocps = []
    for t in range(7):
        sl = pl.ds(rbase + t * C, C)
        ocps.append(pltpu.async_copy(acc.at[sl], out_hbm.at[cid, sl], gs0))
        ocps.append(pltpu.async_copy(accd.at[sl], outd_hbm.at[cid, sl], gs0))
    sl = pl.ds(rbase + 7 * C, 65)
    ocps.append(pltpu.async_copy(acc.at[sl], out_hbm.at[cid, sl], gs0))
    ocps.append(pltpu.async_copy(accd.at[sl], outd_hbm.at[cid, sl], gs0))
    for cp in ocps:
        cp.wait()


def _edge_pass(q, k, v, idx_packed, webe):
    mesh = plsc.VectorSubcoreMesh(core_axis_name="c", subcore_axis_name="s")
    f = pl.kernel(
        _edge_body,
        out_type=[
            jax.ShapeDtypeStruct((NC, N, D), jnp.float32),
            jax.ShapeDtypeStruct((NC, N, 16), jnp.float32),
        ],
        mesh=mesh,
        compiler_params=pltpu.CompilerParams(
            needs_layout_passes=False, use_tc_tiling_on_sc=False),
        scratch_types=[
            pltpu.VMEM_SHARED((N, D), jnp.float32),
            pltpu.VMEM_SHARED((N, 16), jnp.float32),
            pltpu.VMEM((2, 2, 3, C), jnp.int32),
            pltpu.VMEM((2, C, D), jnp.bfloat16),
            pltpu.VMEM((C, D), jnp.bfloat16),
            pltpu.VMEM((2, C, D), jnp.float32),
            pltpu.VMEM((2, C, 16), jnp.float32),
            pltpu.VMEM((2, 16), jnp.float32),
            pltpu.SemaphoreType.DMA,
            pltpu.SemaphoreType.DMA,
            pltpu.SemaphoreType.DMA,
            pltpu.SemaphoreType.DMA,
        ],
    )
    return f(q, k, v, idx_packed, webe)


def _ln(x, g, b, eps=1e-5):
    mu = jnp.mean(x, axis=-1, keepdims=True)
    var = jnp.mean((x - mu) * (x - mu), axis=-1, keepdims=True)
    return (x - mu) / jnp.sqrt(var + eps) * g + b


def _final_body(x_ref, so_ref, p_ref, pd_ref, g1, b1, g2, b2,
                wf1, bf1, wf2, bf2, out_ref):
    num = p_ref[0] + p_ref[1]
    den = pd_ref[0] + pd_ref[1]
    agg = num / jnp.clip(den[:, 0:1], 1e-12, None)
    h = _ln(x_ref[...] + agg + so_ref[...], g1[...], b1[...])
    f = jax.nn.gelu(h @ wf1[...] + bf1[...]) @ wf2[...] + bf2[...]
    out_ref[...] = _ln(h + f, g2[...], b2[...])


def _final(x, selfo, parts, partsd, g1, b1, g2, b2, Wf1, bf1, Wf2, bf2):
    RB = 1000
    grid = (N // RB,)
    row_spec = pl.BlockSpec((RB, D), lambda i: (i, 0))
    p_spec = pl.BlockSpec((NC, RB, D), lambda i: (0, i, 0))
    pd_spec = pl.BlockSpec((NC, RB, 16), lambda i: (0, i, 0))
    vec_spec = pl.BlockSpec((1, D), lambda i: (0, 0))
    vec2_spec = pl.BlockSpec((1, 2 * D), lambda i: (0, 0))
    w1_spec = pl.BlockSpec((D, 2 * D), lambda i: (0, 0))
    w2_spec = pl.BlockSpec((2 * D, D), lambda i: (0, 0))
    return pl.pallas_call(
        _final_body,
        grid=grid,
        in_specs=[row_spec, row_spec, p_spec, pd_spec,
                  vec_spec, vec_spec, vec_spec, vec_spec,
                  w1_spec, vec2_spec, w2_spec, vec_spec],
        out_specs=row_spec,
        out_shape=jax.ShapeDtypeStruct((N, D), jnp.float32),
    )(x, selfo, parts, partsd,
      g1.reshape(1, D), b1.reshape(1, D), g2.reshape(1, D), b2.reshape(1, D),
      Wf1, bf1.reshape(1, 2 * D), Wf2, bf2.reshape(1, D))


def kernel(x, edge_index, edge_weight, Wq, bq, Wk, bk, Wv, bv, Wself, bself,
           We, be, g1, b1, g2, b2, Wf1, bf1, Wf2, bf2):
    src = edge_index[0].astype(jnp.int32)
    dst = edge_index[1].astype(jnp.int32)
    ew = edge_weight.astype(jnp.float32)
    ew_bits = lax.bitcast_convert_type(ew, jnp.int32)

    idx_packed = jnp.stack(
        [src.reshape(NW * NCHUNK, C),
         dst.reshape(NW * NCHUNK, C),
         ew_bits.reshape(NW * NCHUNK, C)], axis=1)
    idx_packed = jnp.concatenate(
        [idx_packed, jnp.zeros((1, 3, C), jnp.int32)], axis=0)

    q, k, v, selfo = _projections(x, Wq, bq, Wk, bk, Wv, bv, Wself, bself)

    webe = jnp.stack([
        jnp.broadcast_to(We.reshape(()), (16,)),
        jnp.broadcast_to(be.reshape(()), (16,)),
    ]).astype(jnp.float32)

    parts, partsd = _edge_pass(q, k, v, idx_packed, webe)

    return _final(x, selfo, parts, partsd,
                  g1, b1, g2, b2, Wf1, bf1, Wf2, bf2)


# speedup vs baseline: 1.0401x; 1.0000x over previous
import jax
import jax.numpy as jnp
from jax import lax
from jax.experimental import pallas as pl
from jax.experimental.pallas import tpu as pltpu
from jax.experimental.pallas import tpu_sc as plsc

N = 10000
E = 320000
D = 128

NC = 2
NS = 16
NW = NC * NS
EPW = E // NW
C = 80
NCHUNK = EPW // C
RPT = N // NS


def _proj_body(x_ref, wq, bq, wk, bk, wv, bv, ws, bs, q_out, k_out, v_out,
               s_out):
    xb = x_ref[...]
    scale = jnp.float32(D ** (-0.5))
    q_out[...] = ((xb @ wq[...] + bq[...]) * scale).astype(jnp.bfloat16)
    k_out[...] = (xb @ wk[...] + bk[...]).astype(jnp.bfloat16)
    v_out[...] = xb @ wv[...] + bv[...]
    s_out[...] = xb @ ws[...] + bs[...]


def _projections(x, Wq, bq, Wk, bk, Wv, bv, Wself, bself):
    RB = 1000
    grid = (N // RB,)
    row_spec = pl.BlockSpec((RB, D), lambda i: (i, 0))
    w_spec = pl.BlockSpec((D, D), lambda i: (0, 0))
    b_spec = pl.BlockSpec((1, D), lambda i: (0, 0))
    return pl.pallas_call(
        _proj_body,
        grid=grid,
        in_specs=[row_spec, w_spec, b_spec, w_spec, b_spec, w_spec, b_spec,
                  w_spec, b_spec],
        out_specs=[row_spec, row_spec, row_spec, row_spec],
        out_shape=[
            jax.ShapeDtypeStruct((N, D), jnp.bfloat16),
            jax.ShapeDtypeStruct((N, D), jnp.bfloat16),
            jax.ShapeDtypeStruct((N, D), jnp.float32),
            jax.ShapeDtypeStruct((N, D), jnp.float32),
        ],
    )(x, Wq, bq.reshape(1, D), Wk, bk.reshape(1, D), Wv, bv.reshape(1, D),
      Wself, bself.reshape(1, D))


def _edge_body(q_hbm, k_hbm, v_hbm, idx_hbm, webe_hbm,
               out_hbm, outd_hbm,
               acc, accd, idx3, qb, kb, vb, obd, webev,
               gs0, gs1, ks, isem):
    cid = lax.axis_index("c")
    sid = lax.axis_index("s")
    wid = sid * NC + cid
    gsems = (gs0, gs1)

    def zrow(i, carry):
        for j in range(D // 16):
            vb[0, i, pl.ds(j * 16, 16)] = jnp.zeros((16,), jnp.float32)
        obd[0, i, :] = jnp.zeros((16,), jnp.float32)
        return carry
    lax.fori_loop(0, C, zrow, 0)

    rbase = sid * RPT
    zcps = []
    for t in range(7):
        zcps.append(pltpu.async_copy(
            vb.at[0], acc.at[pl.ds(rbase + t * C, C)], gs0))
        zcps.append(pltpu.async_copy(
            obd.at[0], accd.at[pl.ds(rbase + t * C, C)], gs0))
    zcps.append(pltpu.async_copy(
        vb.at[0, pl.ds(0, 65)], acc.at[pl.ds(rbase + 7 * C, 65)], gs0))
    zcps.append(pltpu.async_copy(
        obd.at[0, pl.ds(0, 65)], accd.at[pl.ds(rbase + 7 * C, 65)], gs0))
    for cp in zcps:
        cp.wait()
    plsc.subcore_barrier()

    pltpu.sync_copy(webe_hbm, webev)
    we_v = webev[0, :]
    be_v = webev[1, :]
    lane0 = lax.iota(jnp.int32, 16) == 0

    cbase = wid * NCHUNK

    def issue_idx(t, pair_no):
        pltpu.async_copy(idx_hbm.at[pl.ds(cbase + 2 * pair_no, 2)],
                         idx3.at[t], isem)

    def wait_idx(t, pair_no):
        pltpu.make_async_copy(idx_hbm.at[pl.ds(cbase + 2 * pair_no, 2)],
                              idx3.at[t], isem).wait()

    def issue_qv(s, t, r):
        pltpu.async_copy(q_hbm.at[idx3.at[t, r, 1]], qb.at[s], gsems[s])
        pltpu.async_copy(v_hbm.at[idx3.at[t, r, 0]], vb.at[s], gsems[s])

    def wait_qv(s, t, r):
        pltpu.make_async_copy(q_hbm.at[idx3.at[t, r, 1]], qb.at[s],
                              gsems[s]).wait()
        pltpu.make_async_copy(v_hbm.at[idx3.at[t, r, 0]], vb.at[s],
                              gsems[s]).wait()

    def issue_k(t, r):
        pltpu.async_copy(k_hbm.at[idx3.at[t, r, 0]], kb, ks)

    def wait_k(t, r):
        pltpu.make_async_copy(k_hbm.at[idx3.at[t, r, 0]], kb, ks).wait()

    def scores_and_scale(s, t, r):
        def grp(g, carry):
            ewi = idx3[t, r, 2, pl.ds(g * 16, 16)]
            bias16 = plsc.bitcast(ewi, jnp.float32) * we_v + be_v
            for u in range(16):
                e = g * 16 + u
                a = None
                for h in range(4):
                    q32 = qb[s, e, pl.ds(h * 32, 32)]
                    k32 = kb[e, pl.ds(h * 32, 32)]
                    term = q32 * k32
                    a = term if a is None else a + term
                aa, az = plsc.unpack(a, format=plsc.PackFormat.INTERLEAVED)
                sc = jnp.sum(aa + az) + bias16[u]
                exv = jnp.exp(jnp.broadcast_to(sc, (16,)))
                for w in range(D // 16):
                    vb[s, e, pl.ds(w * 16, 16)] = (
                        vb[s, e, pl.ds(w * 16, 16)] * exv)
                obd[s, e, :] = jnp.where(lane0, exv, jnp.float32(0.0))
            return carry
        lax.fori_loop(0, C // 16, grp, 0)

    def scatter(s, t, r):
        pltpu.sync_copy(vb.at[s], acc.at[idx3.at[t, r, 1]], add=True)
        pltpu.sync_copy(obd.at[s], accd.at[idx3.at[t, r, 1]], add=True)

    pltpu.sync_copy(idx_hbm.at[pl.ds(cbase, 2)], idx3.at[0])
    issue_qv(0, 0, 0)
    issue_k(0, 0)

    def quad(i, carry):
        issue_idx(1, 2 * i + 1)
        issue_qv(1, 0, 1)
        wait_qv(0, 0, 0)
        wait_k(0, 0)
        scores_and_scale(0, 0, 0)
        issue_k(0, 1)
        scatter(0, 0, 0)
        wait_idx(1, 2 * i + 1)
        issue_qv(0, 1, 0)
        wait_qv(1, 0, 1)
        wait_k(0, 1)
        scores_and_scale(1, 0, 1)
        issue_k(1, 0)
        scatter(1, 0, 1)
        issue_idx(0, 2 * i + 2)
        issue_qv(1, 1, 1)
        wait_qv(0, 1, 0)
        wait_k(1, 0)
        scores_and_scale(0, 1, 0)
        issue_k(1, 1)
        scatter(0, 1, 0)
        wait_idx(0, 2 * i + 2)
        issue_qv(0, 0, 0)
        wait_qv(1, 1, 1)
        wait_k(1, 1)
        scores_and_scale(1, 1, 1)
        issue_k(0, 0)
        scatter(1, 1, 1)
        return carry
    lax.fori_loop(0, (NCHUNK - 1) // 4, quad, 0)

    wait_qv(0, 0, 0)
    wait_k(0, 0)
    scores_and_scale(0, 0, 0)
    scatter(0, 0, 0)

    plsc.subcore_barrier()
    ocps = []
    for t in range(7):
        sl = pl.ds(rbase + t * C, C)
        ocps.append(pltpu.async_copy(acc.at[sl], out_hbm.at[cid, sl], gs0))
        ocps.append(pltpu.async_copy(accd.at[sl], outd_hbm.at[cid, sl], gs0))
    sl = pl.ds(rbase + 7 * C, 65)
    ocps.append(pltpu.async_copy(acc.at[sl], out_hbm.at[cid, sl], gs0))
    ocps.append(pltpu.async_copy(accd.at[sl], outd_hbm.at[cid, sl], gs0))
    for cp in ocps:
        cp.wait()


def _edge_pass(q, k, v, idx_packed, webe):
    mesh = plsc.VectorSubcoreMesh(core_axis_name="c", subcore_axis_name="s")
    f = pl.kernel(
        _edge_body,
        out_type=[
            jax.ShapeDtypeStruct((NC, N, D), jnp.float32),
            jax.ShapeDtypeStruct((NC, N, 16), jnp.float32),
        ],
        mesh=mesh,
        compiler_params=pltpu.CompilerParams(
            needs_layout_passes=False, use_tc_tiling_on_sc=False),
        scratch_types=[
            pltpu.VMEM_SHARED((N, D), jnp.float32),
            pltpu.VMEM_SHARED((N, 16), jnp.float32),
            pltpu.VMEM((2, 2, 3, C), jnp.int32),
            pltpu.VMEM((2, C, D), jnp.bfloat16),
            pltpu.VMEM((C, D), jnp.bfloat16),
            pltpu.VMEM((2, C, D), jnp.float32),
            pltpu.VMEM((2, C, 16), jnp.float32),
            pltpu.VMEM((2, 16), jnp.float32),
            pltpu.SemaphoreType.DMA,
            pltpu.SemaphoreType.DMA,
            pltpu.SemaphoreType.DMA,
            pltpu.SemaphoreType.DMA,
        ],
    )
    return f(q, k, v, idx_packed, webe)


def _ln(x, g, b, eps=1e-5):
    mu = jnp.mean(x, axis=-1, keepdims=True)
    var = jnp.mean((x - mu) * (x - mu), axis=-1, keepdims=True)
    return (x - mu) / jnp.sqrt(var + eps) * g + b


def _final_body(x_ref, so_ref, p_ref, pd_ref, g1, b1, g2, b2,
                wf1, bf1, wf2, bf2, out_ref):
    num = p_ref[0] + p_ref[1]
    den = pd_ref[0] + pd_ref[1]
    agg = num / jnp.clip(den[:, 0:1], 1e-12, None)
    h = _ln(x_ref[...] + agg + so_ref[...], g1[...], b1[...])
    f = jax.nn.gelu(h @ wf1[...] + bf1[...]) @ wf2[...] + bf2[...]
    out_ref[...] = _ln(h + f, g2[...], b2[...])


def _final(x, selfo, parts, partsd, g1, b1, g2, b2, Wf1, bf1, Wf2, bf2):
    RB = 1000
    grid = (N // RB,)
    row_spec = pl.BlockSpec((RB, D), lambda i: (i, 0))
    p_spec = pl.BlockSpec((NC, RB, D), lambda i: (0, i, 0))
    pd_spec = pl.BlockSpec((NC, RB, 16), lambda i: (0, i, 0))
    vec_spec = pl.BlockSpec((1, D), lambda i: (0, 0))
    vec2_spec = pl.BlockSpec((1, 2 * D), lambda i: (0, 0))
    w1_spec = pl.BlockSpec((D, 2 * D), lambda i: (0, 0))
    w2_spec = pl.BlockSpec((2 * D, D), lambda i: (0, 0))
    return pl.pallas_call(
        _final_body,
        grid=grid,
        in_specs=[row_spec, row_spec, p_spec, pd_spec,
                  vec_spec, vec_spec, vec_spec, vec_spec,
                  w1_spec, vec2_spec, w2_spec, vec_spec],
        out_specs=row_spec,
        out_shape=jax.ShapeDtypeStruct((N, D), jnp.float32),
    )(x, selfo, parts, partsd,
      g1.reshape(1, D), b1.reshape(1, D), g2.reshape(1, D), b2.reshape(1, D),
      Wf1, bf1.reshape(1, 2 * D), Wf2, bf2.reshape(1, D))


def kernel(x, edge_index, edge_weight, Wq, bq, Wk, bk, Wv, bv, Wself, bself,
           We, be, g1, b1, g2, b2, Wf1, bf1, Wf2, bf2):
    src = edge_index[0].astype(jnp.int32)
    dst = edge_index[1].astype(jnp.int32)
    ew = edge_weight.astype(jnp.float32)
    ew_bits = lax.bitcast_convert_type(ew, jnp.int32)

    idx_packed = jnp.stack(
        [src.reshape(NW * NCHUNK, C),
         dst.reshape(NW * NCHUNK, C),
         ew_bits.reshape(NW * NCHUNK, C)], axis=1)
    idx_packed = jnp.concatenate(
        [idx_packed, jnp.zeros((1, 3, C), jnp.int32)], axis=0)

    q, k, v, selfo = _projections(x, Wq, bq, Wk, bk, Wv, bv, Wself, bself)

    webe = jnp.stack([
        jnp.broadcast_to(We.reshape(()), (16,)),
        jnp.broadcast_to(be.reshape(()), (16,)),
    ]).astype(jnp.float32)

    parts, partsd = _edge_pass(q, k, v, idx_packed, webe)

    return _final(x, selfo, parts, partsd,
                  g1, b1, g2, b2, Wf1, bf1, Wf2, bf2)


# speedup vs baseline: 1.0505x; 1.0101x over previous
import jax
import jax.numpy as jnp
from jax import lax
from jax.experimental import pallas as pl
from jax.experimental.pallas import tpu as pltpu
from jax.experimental.pallas import tpu_sc as plsc

N = 10000
E = 320000
D = 128

NC = 2
NS = 16
NW = NC * NS
EPW = E // NW
C = 80
NCHUNK = EPW // C
RPT = N // NS


def _proj_body(x_ref, wq, bq, wk, bk, wv, bv, ws, bs, q_out, k_out, v_out,
               s_out):
    xb = x_ref[...]
    scale = jnp.float32(D ** (-0.5))
    q_out[...] = ((xb @ wq[...] + bq[...]) * scale).astype(jnp.bfloat16)
    k_out[...] = (xb @ wk[...] + bk[...]).astype(jnp.bfloat16)
    v_out[...] = xb @ wv[...] + bv[...]
    s_out[...] = xb @ ws[...] + bs[...]


def _projections(x, Wq, bq, Wk, bk, Wv, bv, Wself, bself):
    RB = 2000
    grid = (N // RB,)
    row_spec = pl.BlockSpec((RB, D), lambda i: (i, 0))
    w_spec = pl.BlockSpec((D, D), lambda i: (0, 0))
    b_spec = pl.BlockSpec((1, D), lambda i: (0, 0))
    return pl.pallas_call(
        _proj_body,
        grid=grid,
        in_specs=[row_spec, w_spec, b_spec, w_spec, b_spec, w_spec, b_spec,
                  w_spec, b_spec],
        out_specs=[row_spec, row_spec, row_spec, row_spec],
        out_shape=[
            jax.ShapeDtypeStruct((N, D), jnp.bfloat16),
            jax.ShapeDtypeStruct((N, D), jnp.bfloat16),
            jax.ShapeDtypeStruct((N, D), jnp.float32),
            jax.ShapeDtypeStruct((N, D), jnp.float32),
        ],
    )(x, Wq, bq.reshape(1, D), Wk, bk.reshape(1, D), Wv, bv.reshape(1, D),
      Wself, bself.reshape(1, D))


def _edge_body(q_hbm, k_hbm, v_hbm, idx_hbm, webe_hbm,
               out_hbm, outd_hbm,
               acc, accd, idx3, qb, kb, vb, obd, webev,
               gs0, gs1, ks, isem):
    cid = lax.axis_index("c")
    sid = lax.axis_index("s")
    wid = sid * NC + cid
    gsems = (gs0, gs1)

    def zrow(i, carry):
        for j in range(D // 16):
            vb[0, i, pl.ds(j * 16, 16)] = jnp.zeros((16,), jnp.float32)
        obd[0, i, :] = jnp.zeros((16,), jnp.float32)
        return carry
    lax.fori_loop(0, C, zrow, 0)

    rbase = sid * RPT
    zcps = []
    for t in range(7):
        zcps.append(pltpu.async_copy(
            vb.at[0], acc.at[pl.ds(rbase + t * C, C)], gs0))
        zcps.append(pltpu.async_copy(
            obd.at[0], accd.at[pl.ds(rbase + t * C, C)], gs0))
    zcps.append(pltpu.async_copy(
        vb.at[0, pl.ds(0, 65)], acc.at[pl.ds(rbase + 7 * C, 65)], gs0))
    zcps.append(pltpu.async_copy(
        obd.at[0, pl.ds(0, 65)], accd.at[pl.ds(rbase + 7 * C, 65)], gs0))
    for cp in zcps:
        cp.wait()
    plsc.subcore_barrier()

    pltpu.sync_copy(webe_hbm, webev)
    we_v = webev[0, :]
    be_v = webev[1, :]
    lane0 = lax.iota(jnp.int32, 16) == 0

    cbase = wid * NCHUNK

    def issue_idx(t, pair_no):
        pltpu.async_copy(idx_hbm.at[pl.ds(cbase + 2 * pair_no, 2)],
                         idx3.at[t], isem)

    def wait_idx(t, pair_no):
        pltpu.make_async_copy(idx_hbm.at[pl.ds(cbase + 2 * pair_no, 2)],
                              idx3.at[t], isem).wait()

    def issue_qv(s, t, r):
        pltpu.async_copy(q_hbm.at[idx3.at[t, r, 1]], qb.at[s], gsems[s])
        pltpu.async_copy(v_hbm.at[idx3.at[t, r, 0]], vb.at[s], gsems[s])

    def wait_qv(s, t, r):
        pltpu.make_async_copy(q_hbm.at[idx3.at[t, r, 1]], qb.at[s],
                              gsems[s]).wait()
        pltpu.make_async_copy(v_hbm.at[idx3.at[t, r, 0]], vb.at[s],
                              gsems[s]).wait()

    def issue_k(t, r):
        pltpu.async_copy(k_hbm.at[idx3.at[t, r, 0]], kb, ks)

    def wait_k(t, r):
        pltpu.make_async_copy(k_hbm.at[idx3.at[t, r, 0]], kb, ks).wait()

    def scores_and_scale(s, t, r):
        def grp(g, carry):
            ewi = idx3[t, r, 2, pl.ds(g * 16, 16)]
            bias16 = plsc.bitcast(ewi, jnp.float32) * we_v + be_v
            for u in range(16):
                e = g * 16 + u
                a = None
                for h in range(4):
                    q32 = qb[s, e, pl.ds(h * 32, 32)]
                    k32 = kb[e, pl.ds(h * 32, 32)]
                    term = q32 * k32
                    a = term if a is None else a + term
                aa, az = plsc.unpack(a, format=plsc.PackFormat.INTERLEAVED)
                sc = jnp.sum(aa + az) + bias16[u]
                exv = jnp.exp(jnp.broadcast_to(sc, (16,)))
                for w in range(D // 16):
                    vb[s, e, pl.ds(w * 16, 16)] = (
                        vb[s, e, pl.ds(w * 16, 16)] * exv)
                obd[s, e, :] = jnp.where(lane0, exv, jnp.float32(0.0))
            return carry
        lax.fori_loop(0, C // 16, grp, 0)

    def scatter(s, t, r):
        pltpu.sync_copy(vb.at[s], acc.at[idx3.at[t, r, 1]], add=True)
        pltpu.sync_copy(obd.at[s], accd.at[idx3.at[t, r, 1]], add=True)

    pltpu.sync_copy(idx_hbm.at[pl.ds(cbase, 2)], idx3.at[0])
    issue_qv(0, 0, 0)
    issue_k(0, 0)

    def quad(i, carry):
        issue_idx(1, 2 * i + 1)
        issue_qv(1, 0, 1)
        wait_qv(0, 0, 0)
        wait_k(0, 0)
        scores_and_scale(0, 0, 0)
        issue_k(0, 1)
        scatter(0, 0, 0)
        wait_idx(1, 2 * i + 1)
        issue_qv(0, 1, 0)
        wait_qv(1, 0, 1)
        wait_k(0, 1)
        scores_and_scale(1, 0, 1)
        issue_k(1, 0)
        scatter(1, 0, 1)
        issue_idx(0, 2 * i + 2)
        issue_qv(1, 1, 1)
        wait_qv(0, 1, 0)
        wait_k(1, 0)
        scores_and_scale(0, 1, 0)
        issue_k(1, 1)
        scatter(0, 1, 0)
        wait_idx(0, 2 * i + 2)
        issue_qv(0, 0, 0)
        wait_qv(1, 1, 1)
        wait_k(1, 1)
        scores_and_scale(1, 1, 1)
        issue_k(0, 0)
        scatter(1, 1, 1)
        return carry
    lax.fori_loop(0, (NCHUNK - 1) // 4, quad, 0)

    wait_qv(0, 0, 0)
    wait_k(0, 0)
    scores_and_scale(0, 0, 0)
    scatter(0, 0, 0)

    plsc.subcore_barrier()
    ocps = []
    for t in range(7):
        sl = pl.ds(rbase + t * C, C)
        ocps.append(pltpu.async_copy(acc.at[sl], out_hbm.at[cid, sl], gs0))
        ocps.append(pltpu.async_copy(accd.at[sl], outd_hbm.at[cid, sl], gs0))
    sl = pl.ds(rbase + 7 * C, 65)
    ocps.append(pltpu.async_copy(acc.at[sl], out_hbm.at[cid, sl], gs0))
    ocps.append(pltpu.async_copy(accd.at[sl], outd_hbm.at[cid, sl], gs0))
    for cp in ocps:
        cp.wait()


def _edge_pass(q, k, v, idx_packed, webe):
    mesh = plsc.VectorSubcoreMesh(core_axis_name="c", subcore_axis_name="s")
    f = pl.kernel(
        _edge_body,
        out_type=[
            jax.ShapeDtypeStruct((NC, N, D), jnp.float32),
            jax.ShapeDtypeStruct((NC, N, 16), jnp.float32),
        ],
        mesh=mesh,
        compiler_params=pltpu.CompilerParams(
            needs_layout_passes=False, use_tc_tiling_on_sc=False),
        scratch_types=[
            pltpu.VMEM_SHARED((N, D), jnp.float32),
            pltpu.VMEM_SHARED((N, 16), jnp.float32),
            pltpu.VMEM((2, 2, 3, C), jnp.int32),
            pltpu.VMEM((2, C, D), jnp.bfloat16),
            pltpu.VMEM((C, D), jnp.bfloat16),
            pltpu.VMEM((2, C, D), jnp.float32),
            pltpu.VMEM((2, C, 16), jnp.float32),
            pltpu.VMEM((2, 16), jnp.float32),
            pltpu.SemaphoreType.DMA,
            pltpu.SemaphoreType.DMA,
            pltpu.SemaphoreType.DMA,
            pltpu.SemaphoreType.DMA,
        ],
    )
    return f(q, k, v, idx_packed, webe)


def _ln(x, g, b, eps=1e-5):
    mu = jnp.mean(x, axis=-1, keepdims=True)
    var = jnp.mean((x - mu) * (x - mu), axis=-1, keepdims=True)
    return (x - mu) / jnp.sqrt(var + eps) * g + b


def _final_body(x_ref, so_ref, p_ref, pd_ref, g1, b1, g2, b2,
                wf1, bf1, wf2, bf2, out_ref):
    num = p_ref[0] + p_ref[1]
    den = pd_ref[0] + pd_ref[1]
    agg = num / jnp.clip(den[:, 0:1], 1e-12, None)
    h = _ln(x_ref[...] + agg + so_ref[...], g1[...], b1[...])
    f = jax.nn.gelu(h @ wf1[...] + bf1[...]) @ wf2[...] + bf2[...]
    out_ref[...] = _ln(h + f, g2[...], b2[...])


def _final(x, selfo, parts, partsd, g1, b1, g2, b2, Wf1, bf1, Wf2, bf2):
    RB = 2000
    grid = (N // RB,)
    row_spec = pl.BlockSpec((RB, D), lambda i: (i, 0))
    p_spec = pl.BlockSpec((NC, RB, D), lambda i: (0, i, 0))
    pd_spec = pl.BlockSpec((NC, RB, 16), lambda i: (0, i, 0))
    vec_spec = pl.BlockSpec((1, D), lambda i: (0, 0))
    vec2_spec = pl.BlockSpec((1, 2 * D), lambda i: (0, 0))
    w1_spec = pl.BlockSpec((D, 2 * D), lambda i: (0, 0))
    w2_spec = pl.BlockSpec((2 * D, D), lambda i: (0, 0))
    return pl.pallas_call(
        _final_body,
        grid=grid,
        in_specs=[row_spec, row_spec, p_spec, pd_spec,
                  vec_spec, vec_spec, vec_spec, vec_spec,
                  w1_spec, vec2_spec, w2_spec, vec_spec],
        out_specs=row_spec,
        out_shape=jax.ShapeDtypeStruct((N, D), jnp.float32),
    )(x, selfo, parts, partsd,
      g1.reshape(1, D), b1.reshape(1, D), g2.reshape(1, D), b2.reshape(1, D),
      Wf1, bf1.reshape(1, 2 * D), Wf2, bf2.reshape(1, D))


def kernel(x, edge_index, edge_weight, Wq, bq, Wk, bk, Wv, bv, Wself, bself,
           We, be, g1, b1, g2, b2, Wf1, bf1, Wf2, bf2):
    src = edge_index[0].astype(jnp.int32)
    dst = edge_index[1].astype(jnp.int32)
    ew = edge_weight.astype(jnp.float32)
    ew_bits = lax.bitcast_convert_type(ew, jnp.int32)

    idx_packed = jnp.stack(
        [src.reshape(NW * NCHUNK, C),
         dst.reshape(NW * NCHUNK, C),
         ew_bits.reshape(NW * NCHUNK, C)], axis=1)
    idx_packed = jnp.concatenate(
        [idx_packed, jnp.zeros((1, 3, C), jnp.int32)], axis=0)

    q, k, v, selfo = _projections(x, Wq, bq, Wk, bk, Wv, bv, Wself, bself)

    webe = jnp.stack([
        jnp.broadcast_to(We.reshape(()), (16,)),
        jnp.broadcast_to(be.reshape(()), (16,)),
    ]).astype(jnp.float32)

    parts, partsd = _edge_pass(q, k, v, idx_packed, webe)

    return _final(x, selfo, parts, partsd,
                  g1, b1, g2, b2, Wf1, bf1, Wf2, bf2)
